# baseline (speedup 1.0000x reference)
import functools

import jax
import jax.numpy as jnp
import numpy as np
from jax import lax
from jax.experimental import pallas as pl
from jax.experimental.pallas import tpu as pltpu
from jax.experimental.pallas import tpu_sc as plsc

_FIELD_DIMS = [38462] * 26
_NUM_FIELDS = 26
_EMBED_DIM = 16
_OFFS = np.array((0, *np.cumsum(_FIELD_DIMS)[:-1]), dtype=np.int32)

_BATCH = 4096
_NW = 32
_BPW = _BATCH // _NW


def _fm_body(idx_hbm, emb_hbm, lin_hbm, bias_hbm, out_hbm,
             idx_v, rows_v, lin_v, ubuf, obuf, bias_v, sem):
    c = lax.axis_index("c")
    s = lax.axis_index("s")
    w = s * 2 + c

    pltpu.sync_copy(idx_hbm.at[w], idx_v)
    pltpu.sync_copy(bias_hbm, bias_v)

    descs = []
    for j in range(_NUM_FIELDS):
        descs.append(pltpu.async_copy(emb_hbm.at[idx_v.at[j]], rows_v.at[j], sem))
        descs.append(pltpu.async_copy(lin_hbm.at[idx_v.at[j]], lin_v.at[j], sem))
    for d in descs:
        d.wait()

    bias16 = bias_v[...]
    lanes = lax.iota(jnp.int32, 16)

    def group_body(g, carry):
        def sample_body(k, carry2):
            b = g * 16 + k
            z = jnp.zeros((16,), jnp.float32)
            s1 = z
            s2 = z
            t1 = z
            t2 = z
            t3 = z
            for j in range(_NUM_FIELDS):
                v0 = rows_v[j, b, pl.ds(0, 16)]
                v1 = rows_v[j, b, pl.ds(16, 16)]
                s1 = s1 + v0
                s2 = s2 + v0 * v0
                q = v1 * v1
                t1 = t1 + v1
                t2 = t2 + q
                t3 = t3 + q * v1
            u = 0.5 * (s1 * s1 - s2) \
                + (t1 * t1 * t1 - 3.0 * t1 * t2 + 2.0 * t3) * (1.0 / 6.0)
            ubuf[pl.ds(k * 16, 16)] = u
            return carry2

        lax.fori_loop(0, 16, sample_body, 0)

        tot = jnp.zeros((16,), jnp.float32)
        rowbase = lanes * 16
        for col in range(16):
            tot = tot + plsc.load_gather(ubuf, [rowbase + col])

        acc = jnp.zeros((16,), jnp.float32)
        for j in range(_NUM_FIELDS):
            acc = acc + lin_v[j, pl.ds(g * 16, 16)]

        y = tot + acc + bias16
        obuf[pl.ds(g * 16, 16)] = 1.0 / (1.0 + jnp.exp(-y))
        return carry

    lax.fori_loop(0, _BPW // 16, group_body, 0)

    pltpu.sync_copy(obuf, out_hbm.at[pl.ds(w * _BPW, _BPW)])


@jax.jit
def _fm_sc(idx3, emb_table, lin1d, bias16):
    mesh = plsc.VectorSubcoreMesh(core_axis_name="c", subcore_axis_name="s")
    f = functools.partial(
        pl.kernel,
        mesh=mesh,
        out_type=jax.ShapeDtypeStruct((_BATCH,), jnp.float32),
        scratch_types=[
            pltpu.VMEM((_NUM_FIELDS, _BPW), jnp.int32),
            pltpu.VMEM((_NUM_FIELDS, _BPW, 2 * _EMBED_DIM), jnp.float32),
            pltpu.VMEM((_NUM_FIELDS, _BPW), jnp.float32),
            pltpu.VMEM((256,), jnp.float32),
            pltpu.VMEM((_BPW,), jnp.float32),
            pltpu.VMEM((16,), jnp.float32),
            pltpu.SemaphoreType.DMA,
        ],
        compiler_params=pltpu.CompilerParams(
            needs_layout_passes=False, use_tc_tiling_on_sc=False),
    )(_fm_body)
    return f(idx3, emb_table, lin1d, bias16)


def kernel(x, emb_table, lin_table, bias):
    xi = x.astype(jnp.int32) + jnp.asarray(_OFFS)[None, :]
    idx3 = xi.reshape(_NW, _BPW, _NUM_FIELDS).transpose(0, 2, 1)
    lin1d = lin_table.reshape(-1)
    bias16 = jnp.broadcast_to(bias.astype(jnp.float32), (16,))
    return _fm_sc(idx3, emb_table, lin1d, bias16)
